# baseline (speedup 1.0000x reference)
import jax
import jax.numpy as jnp
from jax.experimental import pallas as pl
from jax.experimental.pallas import tpu as pltpu

_B, _H, _W, _CIN = 8, 224, 224, 96
_PH, _PW = 16, 16
_NPH, _NPW = 14, 14
_NP = _NPH * _NPW
_KOUT = 256
_M = _B * _PH * _PW


def _reduce_kernel(x_ref, c_ref, o_ref, s_ref, w_ref):
    i = pl.program_id(0)

    @pl.when(i == 0)
    def _init():
        s_ref[...] = jnp.zeros_like(s_ref)
        w_ref[...] = jnp.zeros_like(w_ref)

    acc = x_ref[:, :, 0:_PW, :]
    for j in range(1, _NPW):
        acc = acc + x_ref[:, :, _PW * j:_PW * (j + 1), :]
    s_ref[...] += acc.reshape(_M, _CIN).sum(axis=1, keepdims=True)
    w_ref[...] += c_ref[...].reshape(_NPW * _CIN, _KOUT).sum(axis=0, keepdims=True)

    @pl.when(i == _NPH - 1)
    def _fin():
        o_ref[...] = s_ref[...] * w_ref[...]


def kernel(x, C):
    c3 = C.reshape(_NP, _CIN, _KOUT)
    out2 = pl.pallas_call(
        _reduce_kernel,
        grid=(_NPH,),
        in_specs=[
            pl.BlockSpec((_B, _PH, _W, _CIN), lambda i: (0, i, 0, 0)),
            pl.BlockSpec((_NPW, _CIN, _KOUT), lambda i: (i, 0, 0)),
        ],
        out_specs=pl.BlockSpec((_M, _KOUT), lambda i: (0, 0)),
        out_shape=jax.ShapeDtypeStruct((_M, _KOUT), jnp.float32),
        scratch_shapes=[
            pltpu.VMEM((_M, 1), jnp.float32),
            pltpu.VMEM((1, _KOUT), jnp.float32),
        ],
    )(x, c3)
    out = out2.reshape(_B, _PH, _PW, _KOUT).transpose(0, 3, 1, 2)
    return out.reshape(_B, _KOUT, 1, _PH, _PW)
